# baseline (speedup 1.0000x reference)
import functools

import jax
import jax.numpy as jnp
from jax import lax
from jax.experimental import pallas as pl
from jax.experimental.pallas import tpu as pltpu
from jax.experimental.pallas import tpu_sc as plsc

SEP = 102
DIM = 128
NC, NS = 2, 16
NW = NC * NS
W = 40
LANES = 16
G = 80
WPG = G // W
NBUF = 3
DEPTH = 2


def _prep_body(seq, ids_ref, tt_ref, pos_ref, cidx_ref, add_ref, meta_ref):
    ids = ids_ref[...]
    sep = (ids == SEP).astype(jnp.int32)
    tri = (lax.broadcasted_iota(jnp.int32, (seq, seq), 0)
           <= lax.broadcasted_iota(jnp.int32, (seq, seq), 1)).astype(jnp.bfloat16)
    c = lax.dot_general(sep.astype(jnp.bfloat16), tri,
                        (((1,), (0,)), ((), ())),
                        preferred_element_type=jnp.float32).astype(jnp.int32)
    seg = jnp.minimum(c - sep, 2)
    col = lax.broadcasted_iota(jnp.int32, ids.shape, 1)
    cidx = col + seq * seg
    cidx_ref[...] = cidx

    pos = pos_ref[:seq, :]
    add_ref[:seq, :] = pos + tt_ref[0:1, :]
    add_ref[seq : 2 * seq, :] = pos + tt_ref[1:2, :]
    add_ref[2 * seq : 3 * seq, :] = pos

    for k in range(seq // W):
        first = cidx[:, k * W : k * W + 1]
        last = cidx[:, k * W + W - 1 : k * W + W]
        base = jnp.where(last - first == W - 1, first, -1)
        meta_ref[:, k, :] = jnp.broadcast_to(base, (ids.shape[0], LANES))


def _sc_add_chunks(row_v, a_v, rows):
    avs = []
    for r, s in rows:
        for ch in range(DIM // LANES):
            csl = pl.ds(ch * LANES, LANES)
            avs.append(a_v.at[pl.ds(s, 1), csl][...])
    i = 0
    for r, s in rows:
        for ch in range(DIM // LANES):
            csl = pl.ds(ch * LANES, LANES)
            plsc.addupdate(row_v.at[pl.ds(r, 1), csl], avs[i])
            i += 1


def _gather_body(nwin, table_hbm, addend_hbm, tid_hbm, cidx_hbm, meta_hbm,
                 out_hbm, a_v, tid_v, cid_v, meta_v, row_v, sem_t, sem_w):
    ngath = nwin // WPG
    wid = lax.axis_index("s") * NC + lax.axis_index("c")
    wbase = wid * (nwin * W)
    pltpu.sync_copy(tid_hbm.at[pl.ds(wbase, nwin * W)],
                    tid_v.at[pl.ds(0, nwin * W)])

    def issue_gather(p, g):
        pltpu.async_copy(table_hbm.at[tid_v.at[pl.ds(g * G, G)]],
                         row_v[p], sem_t[p])

    def process(g, p, do_prefetch):
        pltpu.make_async_copy(
            table_hbm.at[tid_v.at[pl.ds(0, G)]], row_v[p], sem_t[p]).wait()

        if do_prefetch:
            q = (p + DEPTH) % NBUF

            @pl.when(g + DEPTH < ngath)
            def _():
                @pl.when(g + DEPTH >= NBUF)
                def _():
                    pltpu.make_async_copy(
                        row_v[q], out_hbm.at[pl.ds(0, G)], sem_w[q]).wait()

                issue_gather(q, g + DEPTH)

        for sub in range(WPG):
            ww = g * WPG + sub
            base = meta_v[pl.ds(ww * LANES, LANES)][0]

            @pl.when(base >= 0)
            def _():
                @pl.loop(0, W, step=4)
                def _(r0):
                    _sc_add_chunks(
                        row_v[p], a_v,
                        [(sub * W + r0 + u, base + r0 + u) for u in range(4)])

            @pl.when(base < 0)
            def _():
                @pl.loop(0, W // 8)
                def _(g8):
                    cvec = cid_v[pl.ds(ww * W + g8 * 8, LANES)]
                    _sc_add_chunks(
                        row_v[p], a_v,
                        [(sub * W + g8 * 8 + j, cvec[j]) for j in range(8)])

        pltpu.async_copy(row_v[p], out_hbm.at[pl.ds(wbase + g * G, G)],
                         sem_w[p])

    for p in range(DEPTH):
        issue_gather(p, p)

    pltpu.sync_copy(addend_hbm, a_v)
    pltpu.sync_copy(cidx_hbm.at[pl.ds(wbase, nwin * W)],
                    cid_v.at[pl.ds(0, nwin * W)])
    pltpu.sync_copy(meta_hbm.at[pl.ds(wid * nwin * LANES, nwin * LANES)],
                    meta_v)

    main = (ngath // NBUF) * NBUF

    @pl.loop(0, main, step=NBUF)
    def _(w):
        for p in range(NBUF):
            process(w + p, p, True)

    for g in range(main, ngath):
        process(g, g % NBUF, False)

    for p in range(NBUF):
        pltpu.make_async_copy(row_v[p], out_hbm.at[pl.ds(0, G)], sem_w[p]).wait()


def kernel(token_ids, token_emb_table, token_type_emb_table, full_position_emb_table):
    batch, seq = token_ids.shape
    token_ids = token_ids.astype(jnp.int32)

    cidx, addend, meta = pl.pallas_call(
        functools.partial(_prep_body, seq),
        out_shape=[
            jax.ShapeDtypeStruct((batch, seq), jnp.int32),
            jax.ShapeDtypeStruct((3 * seq, DIM), jnp.float32),
            jax.ShapeDtypeStruct((batch, seq // W, LANES), jnp.int32),
        ],
    )(token_ids, token_type_emb_table, full_position_emb_table)

    total = batch * seq
    nwin = total // (NW * W)
    tid_flat = token_ids.reshape(-1)
    cid_flat = cidx.reshape(-1)
    meta_flat = meta.reshape(-1)

    mesh = plsc.VectorSubcoreMesh(core_axis_name="c", subcore_axis_name="s")
    out = pl.kernel(
        functools.partial(_gather_body, nwin),
        out_type=jax.ShapeDtypeStruct((total, DIM), jnp.float32),
        mesh=mesh,
        scratch_types=[
            pltpu.VMEM((3 * seq, DIM), jnp.float32),
            pltpu.VMEM((nwin * W,), jnp.int32),
            pltpu.VMEM((nwin * W + LANES,), jnp.int32),
            pltpu.VMEM((nwin * LANES,), jnp.int32),
            [pltpu.VMEM((G, DIM), jnp.float32) for _ in range(NBUF)],
            [pltpu.SemaphoreType.DMA for _ in range(NBUF)],
            [pltpu.SemaphoreType.DMA for _ in range(NBUF)],
        ],
    )(token_emb_table, addend, tid_flat, cid_flat, meta_flat)
    return out.reshape(batch, seq, DIM)


# speedup vs baseline: 1.0452x; 1.0452x over previous
import functools

import jax
import jax.numpy as jnp
from jax import lax
from jax.experimental import pallas as pl
from jax.experimental.pallas import tpu as pltpu
from jax.experimental.pallas import tpu_sc as plsc

SEP = 102
DIM = 128
NC, NS = 2, 16
NW = NC * NS
W = 40
LANES = 16
G = 80
WPG = G // W
NBUF = 3
DEPTH = 2


def _prep_body(seq, ids_ref, tt_ref, pos_ref, cidx_ref, add_ref, meta_ref):
    ids = ids_ref[...]
    sep = (ids == SEP).astype(jnp.int32)
    tri = (lax.broadcasted_iota(jnp.int32, (seq, seq), 0)
           <= lax.broadcasted_iota(jnp.int32, (seq, seq), 1)).astype(jnp.bfloat16)
    c = lax.dot_general(sep.astype(jnp.bfloat16), tri,
                        (((1,), (0,)), ((), ())),
                        preferred_element_type=jnp.float32).astype(jnp.int32)
    seg = jnp.minimum(c - sep, 2)
    col = lax.broadcasted_iota(jnp.int32, ids.shape, 1)
    cidx = col + seq * seg
    cidx_ref[...] = cidx

    pos = pos_ref[:seq, :]
    add_ref[:seq, :] = pos + tt_ref[0:1, :]
    add_ref[seq : 2 * seq, :] = pos + tt_ref[1:2, :]
    add_ref[2 * seq : 3 * seq, :] = pos

    for k in range(seq // W):
        first = cidx[:, k * W : k * W + 1]
        last = cidx[:, k * W + W - 1 : k * W + W]
        base = jnp.where(last - first == W - 1, first, -1)
        meta_ref[:, k, :] = jnp.broadcast_to(base, (ids.shape[0], LANES))


def _sc_add_chunks(row_v, a_v, rows):
    avs = []
    for r, s in rows:
        for ch in range(DIM // LANES):
            csl = pl.ds(ch * LANES, LANES)
            avs.append(a_v.at[pl.ds(s, 1), csl][...])
    i = 0
    for r, s in rows:
        for ch in range(DIM // LANES):
            csl = pl.ds(ch * LANES, LANES)
            plsc.addupdate(row_v.at[pl.ds(r, 1), csl], avs[i])
            i += 1


def _gather_body(nwin, table_hbm, addend_hbm, tid_hbm, cidx_hbm, meta_hbm,
                 out_hbm, a_v, tid_v, cid_v, meta_v, row_v, sem_t, sem_w):
    ngath = nwin // WPG
    wid = lax.axis_index("s") * NC + lax.axis_index("c")
    wbase = wid * (nwin * W)
    pltpu.sync_copy(tid_hbm.at[pl.ds(wbase, nwin * W)],
                    tid_v.at[pl.ds(0, nwin * W)])

    def issue_gather(p, g):
        pltpu.async_copy(table_hbm.at[tid_v.at[pl.ds(g * G, G)]],
                         row_v[p], sem_t[p])

    def process(g, p, do_prefetch):
        pltpu.make_async_copy(
            table_hbm.at[tid_v.at[pl.ds(0, G)]], row_v[p], sem_t[p]).wait()

        for sub in range(WPG):
            ww = g * WPG + sub
            base = meta_v[pl.ds(ww * LANES, LANES)][0]

            @pl.when(base >= 0)
            def _():
                @pl.loop(0, W, step=4)
                def _(r0):
                    _sc_add_chunks(
                        row_v[p], a_v,
                        [(sub * W + r0 + u, base + r0 + u) for u in range(4)])

            @pl.when(base < 0)
            def _():
                @pl.loop(0, W // 8)
                def _(g8):
                    cvec = cid_v[pl.ds(ww * W + g8 * 8, LANES)]
                    _sc_add_chunks(
                        row_v[p], a_v,
                        [(sub * W + g8 * 8 + j, cvec[j]) for j in range(8)])

        pltpu.async_copy(row_v[p], out_hbm.at[pl.ds(wbase + g * G, G)],
                         sem_w[p])

        if do_prefetch:
            q = (p + DEPTH) % NBUF

            @pl.when(g + DEPTH < ngath)
            def _():
                @pl.when(g + DEPTH >= NBUF)
                def _():
                    pltpu.make_async_copy(
                        row_v[q], out_hbm.at[pl.ds(0, G)], sem_w[q]).wait()

                issue_gather(q, g + DEPTH)

    for p in range(DEPTH):
        issue_gather(p, p)

    pltpu.sync_copy(addend_hbm, a_v)
    pltpu.sync_copy(cidx_hbm.at[pl.ds(wbase, nwin * W)],
                    cid_v.at[pl.ds(0, nwin * W)])
    pltpu.sync_copy(meta_hbm.at[pl.ds(wid * nwin * LANES, nwin * LANES)],
                    meta_v)

    main = (ngath // NBUF) * NBUF

    @pl.loop(0, main, step=NBUF)
    def _(w):
        for p in range(NBUF):
            process(w + p, p, True)

    for g in range(main, ngath):
        process(g, g % NBUF, False)

    for p in range(NBUF):
        pltpu.make_async_copy(row_v[p], out_hbm.at[pl.ds(0, G)], sem_w[p]).wait()


def kernel(token_ids, token_emb_table, token_type_emb_table, full_position_emb_table):
    batch, seq = token_ids.shape
    token_ids = token_ids.astype(jnp.int32)

    cidx, addend, meta = pl.pallas_call(
        functools.partial(_prep_body, seq),
        out_shape=[
            jax.ShapeDtypeStruct((batch, seq), jnp.int32),
            jax.ShapeDtypeStruct((3 * seq, DIM), jnp.float32),
            jax.ShapeDtypeStruct((batch, seq // W, LANES), jnp.int32),
        ],
    )(token_ids, token_type_emb_table, full_position_emb_table)

    total = batch * seq
    nwin = total // (NW * W)
    tid_flat = token_ids.reshape(-1)
    cid_flat = cidx.reshape(-1)
    meta_flat = meta.reshape(-1)

    mesh = plsc.VectorSubcoreMesh(core_axis_name="c", subcore_axis_name="s")
    out = pl.kernel(
        functools.partial(_gather_body, nwin),
        out_type=jax.ShapeDtypeStruct((total, DIM), jnp.float32),
        mesh=mesh,
        scratch_types=[
            pltpu.VMEM((3 * seq, DIM), jnp.float32),
            pltpu.VMEM((nwin * W,), jnp.int32),
            pltpu.VMEM((nwin * W + LANES,), jnp.int32),
            pltpu.VMEM((nwin * LANES,), jnp.int32),
            [pltpu.VMEM((G, DIM), jnp.float32) for _ in range(NBUF)],
            [pltpu.SemaphoreType.DMA for _ in range(NBUF)],
            [pltpu.SemaphoreType.DMA for _ in range(NBUF)],
        ],
    )(token_emb_table, addend, tid_flat, cid_flat, meta_flat)
    return out.reshape(batch, seq, DIM)
